# baseline (speedup 1.0000x reference)
import functools

import jax
import jax.numpy as jnp
from jax import lax
from jax.experimental import pallas as pl
from jax.experimental.pallas import tpu as pltpu
from jax.experimental.pallas import tpu_sc as plsc

NU, NB, N = 4000, 6000, 10000
E, D = 320000, 128
NLAYERS = 3
NC, NS = 2, 16
NPAD = 10240
EPT = 10240
CH = 64
NCHUNK = EPT // CH
RPT = NPAD // NS
RB = RPT // CH

_mesh = plsc.VectorSubcoreMesh(core_axis_name="c", subcore_axis_name="s")
_f32 = jnp.float32


def _proj_body(x_ref, wt_ref, b_ref, e_ref, o_ref):
    o_ref[...] = e_ref[...] + b_ref[...] + jnp.dot(
        x_ref[...], wt_ref[...], preferred_element_type=_f32)


def _proj(x, wt, b, etab):
    return pl.pallas_call(
        _proj_body,
        out_shape=jax.ShapeDtypeStruct(etab.shape, _f32),
    )(x, wt, b, etab)


def _hsum_body(h_ref, o_ref):
    o_ref[...] = jnp.sum(h_ref[...].astype(_f32), axis=0)


def _hsum(degH):
    return pl.pallas_call(
        _hsum_body,
        out_shape=jax.ShapeDtypeStruct((NPAD // 16, 16), _f32),
    )(degH)


def _dinv_of(deg):
    return jnp.where(deg > 0.0, lax.rsqrt(deg), 0.0)


def _prep_body(dp_ref, e_ref, g_ref):
    g_ref[...] = _dinv_of(dp_ref[...]) * e_ref[...]


def _prep(degP, emb0p):
    return pl.pallas_call(
        _prep_body,
        out_shape=jax.ShapeDtypeStruct((NPAD, D), _f32),
    )(degP, emb0p)


def _comb_body(p_ref, dp_ref, s_ref, g_ref, so_ref):
    dinv = _dinv_of(dp_ref[...])
    acc = p_ref[0] + p_ref[1]
    emb = dinv * acc
    so_ref[...] = s_ref[...] + emb
    g_ref[...] = dinv * emb


def _comb(P, degP, s):
    return pl.pallas_call(
        _comb_body,
        out_shape=(jax.ShapeDtypeStruct((NPAD, D), _f32),
                   jax.ShapeDtypeStruct((NPAD, D), _f32)),
    )(P, degP, s)


def _comb_last_body(p_ref, dp_ref, s_ref, so_ref):
    dinv = _dinv_of(dp_ref[...])
    emb = dinv * (p_ref[0] + p_ref[1])
    so_ref[...] = (s_ref[...] + emb) * (1.0 / (NLAYERS + 1))


def _comb_last(P, degP, s):
    return pl.pallas_call(
        _comb_last_body,
        out_shape=jax.ShapeDtypeStruct((NPAD, D), _f32),
    )(P, degP, s)


@functools.partial(
    pl.kernel,
    out_type=jax.ShapeDtypeStruct((NC * NS, NPAD // 16, 16), jnp.int32),
    mesh=_mesh,
    compiler_params=pltpu.CompilerParams(needs_layout_passes=False),
    scratch_types=[
        pltpu.VMEM((NPAD // 16, 16), jnp.int32),
        pltpu.VMEM((EPT,), jnp.int32),
    ])
def _deg_kernel(dst_hbm, out_hbm, hist_v, dst_v):
    c = lax.axis_index("c")
    s = lax.axis_index("s")
    w = c * NS + s

    pltpu.sync_copy(dst_hbm.at[w], dst_v)

    def z(i, _):
        hist_v[i, :] = jnp.zeros((16,), jnp.int32)
        return 0
    lax.fori_loop(0, NPAD // 16, z, 0)

    def h(i, _):
        x = dst_v[pl.ds(i * 16, 16)]
        cnt, last = plsc.scan_count(x)
        plsc.addupdate_scatter(hist_v, [x >> 4, x & 15], cnt, mask=last)
        return 0
    lax.fori_loop(0, EPT // 16, h, 0)

    pltpu.sync_copy(hist_v, out_hbm.at[w])


NB = 4
NIS = 16


@functools.partial(
    pl.kernel,
    out_type=jax.ShapeDtypeStruct((NC * NS * RPT, D), _f32),
    mesh=_mesh,
    scratch_types=[
        pltpu.MemorySpace.VMEM_SHARED((NPAD, D), _f32),
        pltpu.VMEM((NIS, CH), jnp.int32),
        pltpu.VMEM((NIS, CH), jnp.int32),
        [pltpu.VMEM((CH, D), _f32) for _ in range(NB)],
        pltpu.SemaphoreType.DMA,
        [pltpu.SemaphoreType.DMA for _ in range(NB)],
        [pltpu.SemaphoreType.DMA for _ in range(NB)],
    ])
def _scat_kernel(g_hbm, src_hbm, dst_hbm, zeros_hbm, out_hbm, acc, src_v,
                 dst_v, rows, sem_i, gsem, ssem):
    c = lax.axis_index("c")
    s = lax.axis_index("s")
    w = c * NS + s
    base = w * NCHUNK

    def idx_copies(j, slot):
        return (pltpu.make_async_copy(src_hbm.at[base + j], src_v.at[slot],
                                      sem_i),
                pltpu.make_async_copy(dst_hbm.at[base + j], dst_v.at[slot],
                                      sem_i))

    for d in idx_copies(0, 0):
        d.start()
    pltpu.sync_copy(zeros_hbm, rows[0])
    zd = [pltpu.async_copy(rows[0], acc.at[pl.ds(s * RPT + k * CH, CH)],
                           ssem[0]) for k in range(RB)]
    for d in zd:
        d.wait()
    plsc.subcore_barrier()

    def gat(j, b):
        return pltpu.make_async_copy(g_hbm.at[src_v.at[j % NIS]], rows[b],
                                     gsem[b])

    def scat_desc(j, b):
        return pltpu.make_async_copy(rows[b], acc.at[dst_v.at[j % NIS]],
                                     ssem[b])

    def body(t, _):
        for b in range(NB):
            j = t * NB + b

            @pl.when(j < NCHUNK - 1)
            def _():
                for d in idx_copies(j + 1, (j + 1) % NIS):
                    d.start()

            @pl.when(j >= NB)
            def _():
                scat_desc(j - NB, b).wait()

            for d in idx_copies(j, j % NIS):
                d.wait()
            gat(j, b).start()

            bm = (b - 1) % NB
            @pl.when(j >= 1)
            def _():
                gat(j - 1, bm).wait()
                pltpu.async_copy(rows[bm], acc.at[dst_v.at[(j - 1) % NIS]],
                                 ssem[bm], add=True)
        return 0
    lax.fori_loop(0, NCHUNK // NB, body, 0)

    j_last = NCHUNK - 1
    b_last = j_last % NB
    gat(j_last, b_last).wait()
    pltpu.async_copy(rows[b_last], acc.at[dst_v.at[j_last % NIS]],
                     ssem[b_last], add=True)
    for b in range(NB):
        scat_desc(j_last - (NB - 1) + b, (b_last + 1 + b) % NB).wait()
    plsc.subcore_barrier()

    w_prev = [None] * NB
    for k in range(RB):
        if w_prev[k % NB] is not None:
            w_prev[k % NB].wait()
        pltpu.async_copy(acc.at[pl.ds(s * RPT + k * CH, CH)], rows[k % NB],
                         gsem[k % NB]).wait()
        w_prev[k % NB] = pltpu.async_copy(
            rows[k % NB], out_hbm.at[pl.ds(w * RPT + k * CH, CH)],
            ssem[k % NB])
    for d in w_prev:
        if d is not None:
            d.wait()


def kernel(edge_index, user_features, book_num_features, book_genre_features,
           emb_table, W_user, b_user, W_bnum, b_bnum, W_bgen, b_bgen):
    src = edge_index[0].astype(jnp.int32)
    dst = edge_index[1].astype(jnp.int32)
    n_pad_e = NC * NS * EPT - E
    pad_idx = N + (jnp.arange(n_pad_e, dtype=jnp.int32) % (NPAD - N))
    src_all = jnp.concatenate([src, pad_idx])
    dst_all = jnp.concatenate([dst, pad_idx])
    src_w = src_all.reshape(NC * NS * NCHUNK, CH)
    dst_w = dst_all.reshape(NC * NS * NCHUNK, CH)
    dst_p = dst_all.reshape(NC * NS, EPT)

    xu = jnp.pad(user_features, ((0, 0), (0, 8)))
    wu = jnp.pad(W_user, ((0, 0), (0, 8))).T
    xb = jnp.concatenate([book_num_features, book_genre_features], axis=1)
    wb = jnp.concatenate([W_bnum, W_bgen], axis=1).T
    bu = b_user[None, :]
    bb = (b_bnum + b_bgen)[None, :]
    emb0_u = _proj(xu, wu, bu, emb_table[:NU])
    emb0_b = _proj(xb, wb, bb, emb_table[NU:])
    emb0 = jnp.concatenate([emb0_u, emb0_b], axis=0)
    emb0p = jnp.pad(emb0, ((0, NPAD - N), (0, 0)))

    zerosD = jnp.zeros((CH, D), _f32)
    degP = _hsum(_deg_kernel(dst_p)).reshape(NPAD, 1)
    g = _prep(degP, emb0p)
    s = emb0p
    for layer in range(NLAYERS):
        P = _scat_kernel(g, src_w, dst_w, zerosD).reshape(NC, NPAD, D)
        if layer < NLAYERS - 1:
            g, s = _comb(P, degP, s)
        else:
            s = _comb_last(P, degP, s)
    return (emb0, s[:N])


# speedup vs baseline: 1.0282x; 1.0282x over previous
import functools

import jax
import jax.numpy as jnp
from jax import lax
from jax.experimental import pallas as pl
from jax.experimental.pallas import tpu as pltpu
from jax.experimental.pallas import tpu_sc as plsc

NU, NB, N = 4000, 6000, 10000
E, D = 320000, 128
NLAYERS = 3
NC, NS = 2, 16
NPAD = 10240
EPT = 10240
CH = 128
NCHUNK = EPT // CH
RPT = NPAD // NS
RB = RPT // CH

_mesh = plsc.VectorSubcoreMesh(core_axis_name="c", subcore_axis_name="s")
_f32 = jnp.float32


_PB = 640


def _projg_body(dp_ref, x_ref, wu_ref, wb_ref, bu_ref, bb_ref, e_ref,
                o_ref, g_ref):
    pid = pl.program_id(0)
    row = pid * _PB + lax.broadcasted_iota(jnp.int32, (_PB, 1), 0)
    x = x_ref[...]
    mu = bu_ref[...] + jnp.dot(x, wu_ref[...], preferred_element_type=_f32)
    mb = bb_ref[...] + jnp.dot(x, wb_ref[...], preferred_element_type=_f32)
    e = e_ref[...] + jnp.where(row < NU, mu, mb)
    o_ref[...] = e
    g_ref[...] = _dinv_of(dp_ref[...]) * e


def _projg(degP, xc, wu, wb, bu, bb, etab_p):
    rs = pl.BlockSpec((_PB, D), lambda i: (i, 0))
    full = lambda shape: pl.BlockSpec(shape, lambda i: (0, 0))
    return pl.pallas_call(
        _projg_body,
        grid=(NPAD // _PB,),
        in_specs=[pl.BlockSpec((_PB, 1), lambda i: (i, 0)),
                  pl.BlockSpec((_PB, 40), lambda i: (i, 0)),
                  full((40, D)), full((40, D)), full((1, D)), full((1, D)),
                  rs],
        out_specs=(rs, rs),
        out_shape=(jax.ShapeDtypeStruct((NPAD, D), _f32),
                   jax.ShapeDtypeStruct((NPAD, D), _f32)),
    )(degP, xc, wu, wb, bu, bb, etab_p)


def _hsum_body(h_ref, o_ref):
    o_ref[...] = jnp.sum(h_ref[...].astype(_f32), axis=0)


def _hsum(degH):
    return pl.pallas_call(
        _hsum_body,
        out_shape=jax.ShapeDtypeStruct((NPAD // 16, 16), _f32),
    )(degH)


def _dinv_of(deg):
    return jnp.where(deg > 0.0, lax.rsqrt(deg), 0.0)


def _comb_body(p_ref, dp_ref, s_ref, g_ref, so_ref):
    dinv = _dinv_of(dp_ref[...])
    acc = p_ref[0] + p_ref[1]
    emb = dinv * acc
    so_ref[...] = s_ref[...] + emb
    g_ref[...] = dinv * emb


def _comb(P, degP, s):
    return pl.pallas_call(
        _comb_body,
        out_shape=(jax.ShapeDtypeStruct((NPAD, D), _f32),
                   jax.ShapeDtypeStruct((NPAD, D), _f32)),
    )(P, degP, s)


def _comb_last_body(p_ref, dp_ref, s_ref, so_ref):
    dinv = _dinv_of(dp_ref[...])
    emb = dinv * (p_ref[0] + p_ref[1])
    so_ref[...] = (s_ref[...] + emb) * (1.0 / (NLAYERS + 1))


def _comb_last(P, degP, s):
    return pl.pallas_call(
        _comb_last_body,
        out_shape=jax.ShapeDtypeStruct((NPAD, D), _f32),
    )(P, degP, s)


@functools.partial(
    pl.kernel,
    out_type=jax.ShapeDtypeStruct((NC * NS, NPAD // 16, 16), jnp.int32),
    mesh=_mesh,
    compiler_params=pltpu.CompilerParams(needs_layout_passes=False),
    scratch_types=[
        pltpu.VMEM((NPAD // 16, 16), jnp.int32),
        pltpu.VMEM((EPT,), jnp.int32),
    ])
def _deg_kernel(dst_hbm, out_hbm, hist_v, dst_v):
    c = lax.axis_index("c")
    s = lax.axis_index("s")
    w = c * NS + s

    pltpu.sync_copy(dst_hbm.at[w], dst_v)

    def z(i, _):
        hist_v[i, :] = jnp.zeros((16,), jnp.int32)
        return 0
    lax.fori_loop(0, NPAD // 16, z, 0)

    def h(i, _):
        x = dst_v[pl.ds(i * 16, 16)]
        cnt, last = plsc.scan_count(x)
        plsc.addupdate_scatter(hist_v, [x >> 4, x & 15], cnt, mask=last)
        return 0
    lax.fori_loop(0, EPT // 16, h, 0)

    pltpu.sync_copy(hist_v, out_hbm.at[w])


NB = 2
NIS = 8


@functools.partial(
    pl.kernel,
    out_type=jax.ShapeDtypeStruct((NC * NS * RPT, D), _f32),
    mesh=_mesh,
    scratch_types=[
        pltpu.MemorySpace.VMEM_SHARED((NPAD, D), _f32),
        pltpu.VMEM((NIS, CH), jnp.int32),
        pltpu.VMEM((NIS, CH), jnp.int32),
        [pltpu.VMEM((CH, D), _f32) for _ in range(NB)],
        pltpu.SemaphoreType.DMA,
        [pltpu.SemaphoreType.DMA for _ in range(NB)],
        [pltpu.SemaphoreType.DMA for _ in range(NB)],
    ])
def _scat_kernel(g_hbm, src_hbm, dst_hbm, zeros_hbm, out_hbm, acc, src_v,
                 dst_v, rows, sem_i, gsem, ssem):
    c = lax.axis_index("c")
    s = lax.axis_index("s")
    w = c * NS + s
    base = w * NCHUNK

    def idx_copies(j, slot):
        return (pltpu.make_async_copy(src_hbm.at[base + j], src_v.at[slot],
                                      sem_i),
                pltpu.make_async_copy(dst_hbm.at[base + j], dst_v.at[slot],
                                      sem_i))

    for d in idx_copies(0, 0):
        d.start()
    pltpu.sync_copy(zeros_hbm, rows[0])
    zd = [pltpu.async_copy(rows[0], acc.at[pl.ds(s * RPT + k * CH, CH)],
                           ssem[0]) for k in range(RB)]
    for d in zd:
        d.wait()
    plsc.subcore_barrier()

    def gat(j, b):
        return pltpu.make_async_copy(g_hbm.at[src_v.at[j % NIS]], rows[b],
                                     gsem[b])

    def scat_desc(j, b):
        return pltpu.make_async_copy(rows[b], acc.at[dst_v.at[j % NIS]],
                                     ssem[b])

    def body(t, _):
        for b in range(NB):
            j = t * NB + b

            @pl.when(j < NCHUNK - 1)
            def _():
                for d in idx_copies(j + 1, (j + 1) % NIS):
                    d.start()

            @pl.when(j >= NB)
            def _():
                scat_desc(j - NB, b).wait()

            for d in idx_copies(j, j % NIS):
                d.wait()
            gat(j, b).start()

            bm = (b - 1) % NB
            @pl.when(j >= 1)
            def _():
                gat(j - 1, bm).wait()
                pltpu.async_copy(rows[bm], acc.at[dst_v.at[(j - 1) % NIS]],
                                 ssem[bm], add=True)
        return 0
    lax.fori_loop(0, NCHUNK // NB, body, 0)

    j_last = NCHUNK - 1
    b_last = j_last % NB
    gat(j_last, b_last).wait()
    pltpu.async_copy(rows[b_last], acc.at[dst_v.at[j_last % NIS]],
                     ssem[b_last], add=True)
    for b in range(NB):
        scat_desc(j_last - (NB - 1) + b, (b_last + 1 + b) % NB).wait()
    plsc.subcore_barrier()

    w_prev = [None] * NB
    for k in range(RB):
        if w_prev[k % NB] is not None:
            w_prev[k % NB].wait()
        pltpu.async_copy(acc.at[pl.ds(s * RPT + k * CH, CH)], rows[k % NB],
                         gsem[k % NB]).wait()
        w_prev[k % NB] = pltpu.async_copy(
            rows[k % NB], out_hbm.at[pl.ds(w * RPT + k * CH, CH)],
            ssem[k % NB])
    for d in w_prev:
        if d is not None:
            d.wait()


def kernel(edge_index, user_features, book_num_features, book_genre_features,
           emb_table, W_user, b_user, W_bnum, b_bnum, W_bgen, b_bgen):
    src = edge_index[0].astype(jnp.int32)
    dst = edge_index[1].astype(jnp.int32)
    n_pad_e = NC * NS * EPT - E
    pad_idx = N + (jnp.arange(n_pad_e, dtype=jnp.int32) % (NPAD - N))
    src_all = jnp.concatenate([src, pad_idx])
    dst_all = jnp.concatenate([dst, pad_idx])
    src_w = src_all.reshape(NC * NS * NCHUNK, CH)
    dst_w = dst_all.reshape(NC * NS * NCHUNK, CH)
    dst_p = dst_all.reshape(NC * NS, EPT)

    xu = jnp.pad(user_features, ((0, 0), (0, 8)))
    wu = jnp.pad(W_user, ((0, 0), (0, 8))).T
    xb = jnp.concatenate([book_num_features, book_genre_features], axis=1)
    wb = jnp.concatenate([W_bnum, W_bgen], axis=1).T
    bu = b_user[None, :]
    bb = (b_bnum + b_bgen)[None, :]
    xc = jnp.pad(jnp.concatenate([xu, xb], axis=0), ((0, NPAD - N), (0, 0)))
    etab_p = jnp.pad(emb_table, ((0, NPAD - N), (0, 0)))

    zerosD = jnp.zeros((CH, D), _f32)
    degP = _hsum(_deg_kernel(dst_p)).reshape(NPAD, 1)
    emb0p, g = _projg(degP, xc, wu, wb, bu, bb, etab_p)
    s = emb0p
    for layer in range(NLAYERS):
        P = _scat_kernel(g, src_w, dst_w, zerosD).reshape(NC, NPAD, D)
        if layer < NLAYERS - 1:
            g, s = _comb(P, degP, s)
        else:
            s = _comb_last(P, degP, s)
    return (emb0p[:N], s[:N])
